# baseline (device time: 21313 ns/iter reference)
import jax
import jax.numpy as jnp
from jax import lax
from jax.experimental import pallas as pl
from jax.experimental.pallas import tpu as pltpu


def kernel(Q, K, V):
    b, q_len, h, d = Q.shape
    k_full = K.shape[1]
    kh = k_full // 2
    hd = h * d
    pack = hd + 2 * h
    scale = d ** -0.5

    Kp = lax.transpose(K, (0, 2, 3, 1))
    Vp = lax.transpose(V, (0, 2, 3, 1))
    Kp = pltpu.with_memory_space_constraint(Kp, pltpu.MemorySpace.HBM)
    Vp = pltpu.with_memory_space_constraint(Vp, pltpu.MemorySpace.HBM)
    Qh = pltpu.with_memory_space_constraint(Q, pltpu.MemorySpace.HBM)

    def body(q_ref, kp_ref, vp_ref, out_ref, qbuf, kbuf, vbuf, pbuf,
             copy_sems, qsem, send_sems, recv_sems):
        my_x = lax.axis_index("x")
        my_y = lax.axis_index("y")
        nbr_y = (my_x, 1 - my_y)
        nbr_x = (1 - my_x, my_y)
        k0 = my_y * kh

        pltpu.make_async_copy(q_ref, qbuf, qsem).start()
        for bi in range(b):
            pltpu.make_async_copy(
                kp_ref.at[bi, :, :, pl.ds(k0, kh)], kbuf.at[bi],
                copy_sems.at[bi, 0]).start()
            pltpu.make_async_copy(
                vp_ref.at[bi, :, :, pl.ds(k0, kh)], vbuf.at[bi],
                copy_sems.at[bi, 1]).start()

        selrows = lax.broadcasted_iota(jnp.int32, (h, hd), 0)
        selcols = lax.broadcasted_iota(jnp.int32, (h, hd), 1)
        selmask = (selcols // d) == selrows
        rrows = lax.broadcasted_iota(jnp.int32, (hd, h), 0)
        rcols = lax.broadcasted_iota(jnp.int32, (hd, h), 1)
        rmask = (rrows // d) == rcols

        pltpu.make_async_copy(q_ref, qbuf, qsem).wait()
        for bi in range(b):
            pltpu.make_async_copy(
                kp_ref.at[bi, :, :, pl.ds(k0, kh)], kbuf.at[bi],
                copy_sems.at[bi, 0]).wait()
            pltpu.make_async_copy(
                vp_ref.at[bi, :, :, pl.ds(k0, kh)], vbuf.at[bi],
                copy_sems.at[bi, 1]).wait()
            q_b = qbuf[bi, 0].astype(jnp.float32)
            w2_b = jnp.where(selmask, jnp.tile(q_b, (1, h)), 0.0)
            kpf_b = kbuf[bi].reshape(hd, kh).astype(jnp.float32)
            vpf_b = vbuf[bi].reshape(hd, kh).astype(jnp.float32)
            s_b = jnp.dot(w2_b, kpf_b,
                          preferred_element_type=jnp.float32) * scale
            m_b = jnp.max(s_b, axis=1, keepdims=True)
            p_b = jnp.exp(s_b - m_b)
            l_b = jnp.sum(p_b, axis=1, keepdims=True)
            r_b = jnp.dot(vpf_b, jnp.transpose(p_b),
                          preferred_element_type=jnp.float32)
            o_b = jnp.sum(jnp.where(rmask, r_b, 0.0), axis=1,
                          keepdims=True)
            pbuf[0, 0:hd, bi:bi + 1] = o_b
            pbuf[0, hd:hd + h, bi:bi + 1] = m_b
            pbuf[0, hd + h:pack, bi:bi + 1] = l_b

        barrier_sem = pltpu.get_barrier_semaphore()
        for nbr in (nbr_y, nbr_x):
            pl.semaphore_signal(barrier_sem, inc=1, device_id=nbr,
                                device_id_type=pl.DeviceIdType.MESH)
        pl.semaphore_wait(barrier_sem, 2)

        def merge(s_a, s_b_, dst, normalize):
            acc0 = pbuf[s_a, 0:hd, :]
            acc1 = pbuf[s_b_, 0:hd, :]
            m0 = pbuf[s_a, hd:hd + h, :]
            m1 = pbuf[s_b_, hd:hd + h, :]
            l0 = pbuf[s_a, hd + h:pack, :]
            l1 = pbuf[s_b_, hd + h:pack, :]
            mn = jnp.maximum(m0, m1)
            a0 = jnp.exp(m0 - mn)
            a1 = jnp.exp(m1 - mn)
            lt = l0 * a0 + l1 * a1
            acc = (acc0 * jnp.repeat(a0, d, axis=0)
                   + acc1 * jnp.repeat(a1, d, axis=0))
            if normalize:
                return acc / jnp.repeat(lt, d, axis=0)
            pbuf[dst, 0:hd, :] = acc
            pbuf[dst, hd:hd + h, :] = mn
            pbuf[dst, hd + h:pack, :] = lt
            return None

        rdma_y = pltpu.make_async_remote_copy(
            src_ref=pbuf.at[0], dst_ref=pbuf.at[1],
            send_sem=send_sems.at[0], recv_sem=recv_sems.at[0],
            device_id=nbr_y, device_id_type=pl.DeviceIdType.MESH,
        )
        rdma_y.start()
        rdma_y.wait()
        merge(0, 1, 2, normalize=False)

        rdma_x = pltpu.make_async_remote_copy(
            src_ref=pbuf.at[2], dst_ref=pbuf.at[3],
            send_sem=send_sems.at[1], recv_sem=recv_sems.at[1],
            device_id=nbr_x, device_id_type=pl.DeviceIdType.MESH,
        )
        rdma_x.start()
        rdma_x.wait()
        of = merge(2, 3, None, normalize=True)

        out_ref[...] = jnp.transpose(of).reshape(b, 1, h, d).astype(out_ref.dtype)

    return pl.pallas_call(
        body,
        out_shape=jax.ShapeDtypeStruct((b, q_len, h, d), jnp.float32),
        in_specs=[
            pl.BlockSpec(memory_space=pltpu.MemorySpace.HBM),
            pl.BlockSpec(memory_space=pltpu.MemorySpace.HBM),
            pl.BlockSpec(memory_space=pltpu.MemorySpace.HBM),
        ],
        out_specs=pl.BlockSpec(memory_space=pltpu.VMEM),
        scratch_shapes=[
            pltpu.VMEM((b, q_len, h, d), jnp.float32),
            pltpu.VMEM((b, h, d, kh), jnp.float32),
            pltpu.VMEM((b, h, d, kh), jnp.float32),
            pltpu.VMEM((4, pack, b), jnp.float32),
            pltpu.SemaphoreType.DMA((b, 2)),
            pltpu.SemaphoreType.DMA,
            pltpu.SemaphoreType.DMA((2,)),
            pltpu.SemaphoreType.DMA((2,)),
        ],
        compiler_params=pltpu.CompilerParams(collective_id=0),
    )(Qh, Kp, Vp)
